# baseline (speedup 1.0000x reference)
import jax
import jax.numpy as jnp
from jax.experimental import pallas as pl

T, H, L, V, B, MAXN = 8192, 512, 128, 1024, 256, 15
R = 256
NB = T // R


def _mm(a, b):
    return jnp.dot(a.astype(jnp.bfloat16), b, preferred_element_type=jnp.float32)


def _body(idx_ref, ctx_ref, hnei_ref, tv_ref, emb_ref,
          wz_ref, wr_ref, ur_ref, wh_ref, w1_ref, w2_ref, wo_ref,
          ui_ref, u1_ref, u2_ref, bias_ref,
          word_ref, stop_ref):
    f32 = jnp.float32
    bf16 = jnp.bfloat16
    idx = idx_ref[0, 0, :]
    ctx = ctx_ref[0, 0, :]

    iota_v = jax.lax.broadcasted_iota(jnp.int32, (R, V), 1)
    oh_x = (idx[:, None] == iota_v).astype(bf16)
    x = jnp.dot(oh_x, emb_ref[...], preferred_element_type=f32)

    iota_b = jax.lax.broadcasted_iota(jnp.int32, (R, B), 1)
    oh_c = (ctx[:, None] == iota_b).astype(bf16)
    tc = jnp.dot(oh_c, tv_ref[...], preferred_element_type=f32)

    wz_b = bias_ref[0, :H]
    wr_b = bias_ref[1, :H]
    wh_b = bias_ref[2, :H]
    w_b = bias_ref[3, :H]
    ui_b = bias_ref[4, :H]
    u_b = bias_ref[5, :H]
    uo_row = bias_ref[6, :H]
    uo_b = bias_ref[7, 0]

    r1 = _mm(x, wr_ref[...]) + wr_b[None, :]
    sum_h = jnp.zeros((R, H), f32)
    sum_gated = jnp.zeros((R, H), f32)
    for n in range(MAXN):
        h = hnei_ref[:, n, :]
        g = jax.nn.sigmoid(r1 + jnp.dot(h.astype(bf16), ur_ref[...],
                                        preferred_element_type=f32))
        sum_h = sum_h + h
        sum_gated = sum_gated + g * h

    z = jax.nn.sigmoid(_mm(x, wz_ref[:H, :]) + _mm(sum_h, wz_ref[H:, :])
                       + wz_b[None, :])
    pre_h = jnp.tanh(_mm(x, wh_ref[:H, :]) + _mm(sum_gated, wh_ref[H:, :])
                     + wh_b[None, :])
    new_h = (1.0 - z) * sum_h + z * pre_h

    wh_act = jax.nn.relu(_mm(new_h, w1_ref[...]) + _mm(tc, w2_ref[...])
                         + w_b[None, :])
    word = _mm(wh_act, wo_ref[...])
    word_ref[...] = word + bias_ref[8:8 + (V // H), :].reshape(1, V)

    sh = jax.nn.relu(_mm(x, ui_ref[:H, :]) + _mm(sum_h, ui_ref[H:, :])
                     + ui_b[None, :])
    sh2 = jax.nn.relu(_mm(sh, u1_ref[...]) + _mm(tc, u2_ref[...])
                      + u_b[None, :])
    stop = jnp.sum(sh2 * uo_row[None, :], axis=1, keepdims=True) + uo_b
    stop_ref[...] = jnp.broadcast_to(stop, (R, 128))


@jax.jit
def _run(cur_x_idx, contexts, cur_h_nei, tree_vecs, emb, Wz_w, Wz_b, Wr_w,
         Wr_b, Ur_w, Wh_w, Wh_b, W_w, W_b, Wo_w, Wo_b, Ui_w, Ui_b, U_w, U_b,
         Uo_w, Uo_b):
    f32 = jnp.float32
    bf16 = jnp.bfloat16
    idx2 = cur_x_idx.astype(jnp.int32).reshape(NB, 1, R)
    ctx2 = contexts.astype(jnp.int32).reshape(NB, 1, R)

    wz = Wz_w.T.astype(bf16)
    wr = Wr_w.T.astype(bf16)
    ur = Ur_w.T.astype(bf16)
    wh = Wh_w.T.astype(bf16)
    w1 = W_w.T[:H, :].astype(bf16)
    w2 = W_w.T[H:, :].astype(bf16)
    wo = Wo_w.T.astype(bf16)
    ui = Ui_w.T.astype(bf16)
    u1 = U_w.T[:H, :].astype(bf16)
    u2 = U_w.T[H:, :].astype(bf16)
    emb_bf = emb.astype(bf16)
    tv_bf = tree_vecs.astype(bf16)

    bias = jnp.stack([
        Wz_b, Wr_b, Wh_b, W_b, Ui_b, U_b, Uo_w[0, :],
        jnp.full((H,), Uo_b[0], f32),
    ], axis=0)
    bias = jnp.concatenate([bias, Wo_b.reshape(V // H, H)], axis=0)

    full = lambda shape: pl.BlockSpec(shape, lambda i: (0,) * len(shape))
    grid = (NB,)
    in_specs = [
        pl.BlockSpec((1, 1, R), lambda i: (i, 0, 0)),
        pl.BlockSpec((1, 1, R), lambda i: (i, 0, 0)),
        pl.BlockSpec((R, MAXN, H), lambda i: (i, 0, 0)),
        full((B, L)),
        full((V, H)),
        full((2 * H, H)),
        full((H, H)),
        full((H, H)),
        full((2 * H, H)),
        full((H, H)),
        full((L, H)),
        full((H, V)),
        full((2 * H, H)),
        full((H, H)),
        full((L, H)),
        full((8 + V // H, H)),
    ]
    out_specs = [
        pl.BlockSpec((R, V), lambda i: (i, 0)),
        pl.BlockSpec((R, 128), lambda i: (i, 0)),
    ]
    word, stop = pl.pallas_call(
        _body,
        grid=grid,
        in_specs=in_specs,
        out_specs=out_specs,
        out_shape=[
            jax.ShapeDtypeStruct((T, V), f32),
            jax.ShapeDtypeStruct((T, 128), f32),
        ],
    )(idx2, ctx2, cur_h_nei, tv_bf, emb_bf, wz, wr, ur, wh, w1, w2, wo,
      ui, u1, u2, bias)
    return jnp.concatenate([word, stop[:, :1]], axis=1)


def kernel(cur_x_idx, contexts, cur_h_nei, tree_vecs, emb, Wz_w, Wz_b, Wr_w,
           Wr_b, Ur_w, Wh_w, Wh_b, W_w, W_b, Wo_w, Wo_b, Ui_w, Ui_b, U_w,
           U_b, Uo_w, Uo_b):
    return _run(cur_x_idx, contexts, cur_h_nei, tree_vecs, emb, Wz_w, Wz_b,
                Wr_w, Wr_b, Ur_w, Wh_w, Wh_b, W_w, W_b, Wo_w, Wo_b, Ui_w,
                Ui_b, U_w, U_b, Uo_w, Uo_b)


# speedup vs baseline: 1.0231x; 1.0231x over previous
import jax
import jax.numpy as jnp
from jax.experimental import pallas as pl
from jax.experimental.pallas import tpu as pltpu

T, H, L, V, B, MAXN = 8192, 512, 128, 1024, 256, 15
R = 128
RN = R * MAXN
NB = T // R
f32 = jnp.float32
bf16 = jnp.bfloat16


def _dgt(a, w):
    return jax.lax.dot_general(a.astype(bf16), w, (((1,), (1,)), ((), ())),
                               preferred_element_type=f32)


def _body(idx_ref, ctx_ref, hnei_ref, tv_ref, emb_ref,
          wz_ref, wr_ref, ur_ref, wh_ref, w_ref, wo_ref,
          ui_ref, u_ref, bias_ref,
          out_ref,
          wz_s, wr_s, ur_s, wh_s, w_s, wo_s, ui_s, u_s, emb_s, tv_s,
          seg_s, segt_s):
    i = pl.program_id(0)

    @pl.when(i == 0)
    def _prep():
        wz_s[...] = wz_ref[...].astype(bf16)
        wr_s[...] = wr_ref[...].astype(bf16)
        ur_s[...] = ur_ref[...].astype(bf16)
        wh_s[...] = wh_ref[...].astype(bf16)
        w_s[...] = w_ref[...].astype(bf16)
        wo_s[...] = wo_ref[...].astype(bf16)
        ui_s[...] = ui_ref[...].astype(bf16)
        u_s[...] = u_ref[...].astype(bf16)
        emb_s[...] = emb_ref[...].astype(bf16)
        tv_s[...] = tv_ref[...].astype(bf16)
        rows = jax.lax.broadcasted_iota(jnp.int32, (R, RN), 0)
        cols = jax.lax.broadcasted_iota(jnp.int32, (R, RN), 1)
        seg_s[...] = (cols - rows * MAXN < MAXN).astype(bf16) * \
                     (cols - rows * MAXN >= 0).astype(bf16)
        rows_t = jax.lax.broadcasted_iota(jnp.int32, (RN, R), 0)
        cols_t = jax.lax.broadcasted_iota(jnp.int32, (RN, R), 1)
        segt_s[...] = (rows_t - cols_t * MAXN < MAXN).astype(bf16) * \
                      (rows_t - cols_t * MAXN >= 0).astype(bf16)

    idx = idx_ref[0, 0, :]
    ctx = ctx_ref[0, 0, :]

    x2 = hnei_ref[...]
    x2b = x2.astype(bf16)

    sum_h = jnp.dot(seg_s[...], x2b, preferred_element_type=f32)

    iota_v = jax.lax.broadcasted_iota(jnp.int32, (R, V), 1)
    oh_x = (idx[:, None] == iota_v).astype(bf16)
    x = jnp.dot(oh_x, emb_s[...], preferred_element_type=f32)

    iota_b = jax.lax.broadcasted_iota(jnp.int32, (R, B), 1)
    oh_c = (ctx[:, None] == iota_b).astype(bf16)
    tc = jnp.dot(oh_c, tv_s[...], preferred_element_type=f32)

    wz_b = bias_ref[0, :H]
    wr_b = bias_ref[1, :H]
    wh_b = bias_ref[2, :H]
    w_b = bias_ref[3, :H]
    ui_b = bias_ref[4, :H]
    u_b = bias_ref[5, :H]
    uo_row = bias_ref[6, :H]
    uo_b = bias_ref[7, 0]

    r1 = _dgt(x, wr_s[...]) + wr_b[None, :]
    r1_full = jnp.dot(segt_s[...], r1.astype(bf16),
                      preferred_element_type=f32)
    r2 = _dgt(x2b, ur_s[...])
    g = jax.nn.sigmoid(r1_full + r2)
    p = (g * x2).astype(bf16)
    sum_gated = jnp.dot(seg_s[...], p, preferred_element_type=f32)

    z = jax.nn.sigmoid(_dgt(x, wz_s[:, :H]) + _dgt(sum_h, wz_s[:, H:])
                       + wz_b[None, :])
    pre_h = jnp.tanh(_dgt(x, wh_s[:, :H]) + _dgt(sum_gated, wh_s[:, H:])
                     + wh_b[None, :])
    new_h = (1.0 - z) * sum_h + z * pre_h

    wh_act = jax.nn.relu(_dgt(new_h, w_s[:, :H]) + _dgt(tc, w_s[:, H:])
                         + w_b[None, :])
    word = _dgt(wh_act, wo_s[...])
    out_ref[:, :V] = word + bias_ref[8:8 + (V // H), :].reshape(1, V)

    sh = jax.nn.relu(_dgt(x, ui_s[:, :H]) + _dgt(sum_h, ui_s[:, H:])
                     + ui_b[None, :])
    sh2 = jax.nn.relu(_dgt(sh, u_s[:, :H]) + _dgt(tc, u_s[:, H:])
                      + u_b[None, :])
    stop = jnp.sum(sh2 * uo_row[None, :], axis=1, keepdims=True) + uo_b
    out_ref[:, V:] = stop


@jax.jit
def _run(cur_x_idx, contexts, cur_h_nei, tree_vecs, emb, Wz_w, Wz_b, Wr_w,
         Wr_b, Ur_w, Wh_w, Wh_b, W_w, W_b, Wo_w, Wo_b, Ui_w, Ui_b, U_w, U_b,
         Uo_w, Uo_b):
    idx2 = cur_x_idx.astype(jnp.int32).reshape(NB, 1, R)
    ctx2 = contexts.astype(jnp.int32).reshape(NB, 1, R)
    hnei2 = cur_h_nei.reshape(T * MAXN, H)

    bias = jnp.stack([
        Wz_b, Wr_b, Wh_b, W_b, Ui_b, U_b, Uo_w[0, :],
        jnp.full((H,), Uo_b[0], f32),
    ], axis=0)
    bias = jnp.concatenate([bias, Wo_b.reshape(V // H, H)], axis=0)

    full = lambda shape: pl.BlockSpec(shape, lambda i: (0,) * len(shape))
    grid = (NB,)
    in_specs = [
        pl.BlockSpec((1, 1, R), lambda i: (i, 0, 0)),
        pl.BlockSpec((1, 1, R), lambda i: (i, 0, 0)),
        pl.BlockSpec((RN, H), lambda i: (i, 0)),
        full((B, L)),
        full((V, H)),
        full((H, 2 * H)),
        full((H, H)),
        full((H, H)),
        full((H, 2 * H)),
        full((H, H + L)),
        full((V, H)),
        full((H, 2 * H)),
        full((H, H + L)),
        full((8 + V // H, H)),
    ]
    out_specs = pl.BlockSpec((R, V + 1), lambda i: (i, 0))
    out = pl.pallas_call(
        _body,
        grid=grid,
        in_specs=in_specs,
        out_specs=out_specs,
        out_shape=jax.ShapeDtypeStruct((T, V + 1), f32),
        scratch_shapes=[
            pltpu.VMEM((H, 2 * H), bf16),
            pltpu.VMEM((H, H), bf16),
            pltpu.VMEM((H, H), bf16),
            pltpu.VMEM((H, 2 * H), bf16),
            pltpu.VMEM((H, H + L), bf16),
            pltpu.VMEM((V, H), bf16),
            pltpu.VMEM((H, 2 * H), bf16),
            pltpu.VMEM((H, H + L), bf16),
            pltpu.VMEM((V, H), bf16),
            pltpu.VMEM((B, L), bf16),
            pltpu.VMEM((R, RN), bf16),
            pltpu.VMEM((RN, R), bf16),
        ],
    )(idx2, ctx2, hnei2, tree_vecs, emb, Wz_w, Wr_w, Ur_w, Wh_w, W_w, Wo_w,
      Ui_w, U_w, bias)
    return out


def kernel(cur_x_idx, contexts, cur_h_nei, tree_vecs, emb, Wz_w, Wz_b, Wr_w,
           Wr_b, Ur_w, Wh_w, Wh_b, W_w, W_b, Wo_w, Wo_b, Ui_w, Ui_b, U_w,
           U_b, Uo_w, Uo_b):
    return _run(cur_x_idx, contexts, cur_h_nei, tree_vecs, emb, Wz_w, Wz_b,
                Wr_w, Wr_b, Ur_w, Wh_w, Wh_b, W_w, W_b, Wo_w, Wo_b, Ui_w,
                Ui_b, U_w, U_b, Uo_w, Uo_b)


# speedup vs baseline: 1.2638x; 1.2353x over previous
import jax
import jax.numpy as jnp
from jax.experimental import pallas as pl
from jax.experimental.pallas import tpu as pltpu

T, H, L, V, B, MAXN = 8192, 512, 128, 1024, 256, 15
R = 128
RN = R * MAXN
NB = T // R
f32 = jnp.float32
bf16 = jnp.bfloat16


def _dgt(a, w):
    return jax.lax.dot_general(a.astype(bf16), w, (((1,), (1,)), ((), ())),
                               preferred_element_type=f32)


def _body(idx_ref, ctx_ref, hnei_ref, tv_ref, emb_ref,
          wz_ref, wr_ref, ur_ref, wh_ref, w_ref, wo_ref,
          ui_ref, u_ref, bias_ref,
          out_ref,
          wz_s, wr_s, ur_s, wh_s, w_s, wo_s, ui_s, u_s, emb_s, tv_s,
          seg_s, segt_s):
    i = pl.program_id(0)

    @pl.when(i == 0)
    def _prime():
        wz_s[...] = wz_ref[...].astype(bf16)
        wr_s[...] = wr_ref[...].astype(bf16)
        ur_s[...] = ur_ref[...].astype(bf16)
        wh_s[...] = wh_ref[...].astype(bf16)
        w_s[...] = w_ref[...].astype(bf16)
        wo_s[...] = wo_ref[...].astype(bf16)
        ui_s[...] = ui_ref[...].astype(bf16)
        u_s[...] = u_ref[...].astype(bf16)
        emb_s[...] = emb_ref[...].astype(bf16)
        tv_s[...] = tv_ref[...].astype(bf16)
        rows = jax.lax.broadcasted_iota(jnp.int32, (R, RN), 0)
        cols = jax.lax.broadcasted_iota(jnp.int32, (R, RN), 1)
        seg_s[...] = (cols // MAXN == rows).astype(bf16)
        rows_t = jax.lax.broadcasted_iota(jnp.int32, (RN, R), 0)
        cols_t = jax.lax.broadcasted_iota(jnp.int32, (RN, R), 1)
        segt_s[...] = (rows_t // MAXN == cols_t).astype(bf16)

    idx = idx_ref[0, 0, :]
    ctx = ctx_ref[0, 0, :]

    x2 = hnei_ref[...].reshape(RN, H)
    x2b = x2.astype(bf16)

    sum_h = jnp.dot(seg_s[...], x2b, preferred_element_type=f32)

    iota_v = jax.lax.broadcasted_iota(jnp.int32, (R, V), 1)
    oh_x = (idx[:, None] == iota_v).astype(bf16)
    x = jnp.dot(oh_x, emb_s[...], preferred_element_type=f32)

    iota_b = jax.lax.broadcasted_iota(jnp.int32, (R, B), 1)
    oh_c = (ctx[:, None] == iota_b).astype(bf16)
    tc = jnp.dot(oh_c, tv_s[...], preferred_element_type=f32)

    wz_b = bias_ref[0, :H]
    wr_b = bias_ref[1, :H]
    wh_b = bias_ref[2, :H]
    w_b = bias_ref[3, :H]
    ui_b = bias_ref[4, :H]
    u_b = bias_ref[5, :H]
    uo_row = bias_ref[6, :H]
    uo_b = bias_ref[7, 0]

    r1 = _dgt(x, wr_s[...]) + wr_b[None, :]
    r1_full = jnp.dot(segt_s[...], r1.astype(bf16),
                      preferred_element_type=f32)
    r2 = _dgt(x2b, ur_s[...])
    g = jax.nn.sigmoid(r1_full + r2)
    p = (g * x2).astype(bf16)
    sum_gated = jnp.dot(seg_s[...], p, preferred_element_type=f32)

    z = jax.nn.sigmoid(_dgt(x, wz_s[:, :H]) + _dgt(sum_h, wz_s[:, H:])
                       + wz_b[None, :])
    pre_h = jnp.tanh(_dgt(x, wh_s[:, :H]) + _dgt(sum_gated, wh_s[:, H:])
                     + wh_b[None, :])
    new_h = (1.0 - z) * sum_h + z * pre_h

    wh_act = jax.nn.relu(_dgt(new_h, w_s[:, :H]) + _dgt(tc, w_s[:, H:])
                         + w_b[None, :])
    word = _dgt(wh_act, wo_s[...])
    out_ref[:, :V] = word + bias_ref[8:8 + (V // H), :].reshape(1, V)

    sh = jax.nn.relu(_dgt(x, ui_s[:, :H]) + _dgt(sum_h, ui_s[:, H:])
                     + ui_b[None, :])
    sh2 = jax.nn.relu(_dgt(sh, u_s[:, :H]) + _dgt(tc, u_s[:, H:])
                      + u_b[None, :])
    stop = jnp.sum(sh2 * uo_row[None, :], axis=1, keepdims=True) + uo_b
    out_ref[:, V:] = stop


@jax.jit
def _run(cur_x_idx, contexts, cur_h_nei, tree_vecs, emb, Wz_w, Wz_b, Wr_w,
         Wr_b, Ur_w, Wh_w, Wh_b, W_w, W_b, Wo_w, Wo_b, Ui_w, Ui_b, U_w, U_b,
         Uo_w, Uo_b):
    idx2 = cur_x_idx.astype(jnp.int32).reshape(NB, 1, R)
    ctx2 = contexts.astype(jnp.int32).reshape(NB, 1, R)

    bias = jnp.stack([
        Wz_b, Wr_b, Wh_b, W_b, Ui_b, U_b, Uo_w[0, :],
        jnp.full((H,), Uo_b[0], f32),
    ], axis=0)
    bias = jnp.concatenate([bias, Wo_b.reshape(V // H, H)], axis=0)

    full = lambda shape: pl.BlockSpec(shape, lambda i: (0,) * len(shape))
    grid = (NB,)
    in_specs = [
        pl.BlockSpec((1, 1, R), lambda i: (i, 0, 0)),
        pl.BlockSpec((1, 1, R), lambda i: (i, 0, 0)),
        pl.BlockSpec((R, MAXN, H), lambda i: (i, 0, 0)),
        full((B, L)),
        full((V, H)),
        full((H, 2 * H)),
        full((H, H)),
        full((H, H)),
        full((H, 2 * H)),
        full((H, H + L)),
        full((V, H)),
        full((H, 2 * H)),
        full((H, H + L)),
        full((8 + V // H, H)),
    ]
    out_specs = pl.BlockSpec((R, V + 1), lambda i: (i, 0))
    out = pl.pallas_call(
        _body,
        grid=grid,
        in_specs=in_specs,
        out_specs=out_specs,
        out_shape=jax.ShapeDtypeStruct((T, V + 1), f32),
        scratch_shapes=[
            pltpu.VMEM((H, 2 * H), bf16),
            pltpu.VMEM((H, H), bf16),
            pltpu.VMEM((H, H), bf16),
            pltpu.VMEM((H, 2 * H), bf16),
            pltpu.VMEM((H, H + L), bf16),
            pltpu.VMEM((V, H), bf16),
            pltpu.VMEM((H, 2 * H), bf16),
            pltpu.VMEM((H, H + L), bf16),
            pltpu.VMEM((V, H), bf16),
            pltpu.VMEM((B, L), bf16),
            pltpu.VMEM((R, RN), bf16),
            pltpu.VMEM((RN, R), bf16),
        ],
    )(idx2, ctx2, cur_h_nei, tree_vecs, emb, Wz_w, Wr_w, Ur_w, Wh_w, W_w,
      Wo_w, Ui_w, U_w, bias)
    return out


def kernel(cur_x_idx, contexts, cur_h_nei, tree_vecs, emb, Wz_w, Wz_b, Wr_w,
           Wr_b, Ur_w, Wh_w, Wh_b, W_w, W_b, Wo_w, Wo_b, Ui_w, Ui_b, U_w,
           U_b, Uo_w, Uo_b):
    return _run(cur_x_idx, contexts, cur_h_nei, tree_vecs, emb, Wz_w, Wz_b,
                Wr_w, Wr_b, Ur_w, Wh_w, Wh_b, W_w, W_b, Wo_w, Wo_b, Ui_w,
                Ui_b, U_w, U_b, Uo_w, Uo_b)


# speedup vs baseline: 1.3202x; 1.0446x over previous
import jax
import jax.numpy as jnp
from jax.experimental import pallas as pl
from jax.experimental.pallas import tpu as pltpu

T, H, L, V, B, MAXN = 8192, 512, 128, 1024, 256, 15
R = 128
RN = R * MAXN
NB = T // R
f32 = jnp.float32
bf16 = jnp.bfloat16


def _dgt(a, w):
    return jax.lax.dot_general(a.astype(bf16), w, (((1,), (1,)), ((), ())),
                               preferred_element_type=f32)


def _body(idx_ref, ctx_ref, hnei_ref, tv_ref, emb_ref,
          wz_ref, wr_ref, ur_ref, wh_ref, w_ref, wo_ref,
          ui_ref, u_ref, bias_ref,
          out_ref,
          wz_s, wr_s, ur_s, wh_s, w_s, wo_s, ui_s, u_s, emb_s, tv_s,
          seg_s, segt_s):
    i = pl.program_id(0)

    @pl.when(i == 0)
    def _prime():
        wz_s[...] = wz_ref[...].astype(bf16)
        wr_s[...] = wr_ref[...].astype(bf16)
        ur_s[...] = ur_ref[...].astype(bf16)
        wh_s[...] = wh_ref[...].astype(bf16)
        w_s[...] = w_ref[...].astype(bf16)
        wo_s[...] = wo_ref[...].astype(bf16)
        ui_s[...] = ui_ref[...].astype(bf16)
        u_s[...] = u_ref[...].astype(bf16)
        emb_s[...] = emb_ref[...].astype(bf16)
        tv_s[...] = tv_ref[...].astype(bf16)
        rows = jax.lax.broadcasted_iota(jnp.int32, (R, RN), 0)
        cols = jax.lax.broadcasted_iota(jnp.int32, (R, RN), 1)
        seg_s[...] = (cols // MAXN == rows).astype(bf16)
        rows_t = jax.lax.broadcasted_iota(jnp.int32, (RN, R), 0)
        cols_t = jax.lax.broadcasted_iota(jnp.int32, (RN, R), 1)
        segt_s[...] = (rows_t // MAXN == cols_t).astype(bf16)

    idx = idx_ref[0, 0, :]
    ctx = ctx_ref[0, 0, :]

    x2b = hnei_ref[...].astype(bf16).reshape(RN, H)

    sum_h = jnp.dot(seg_s[...], x2b, preferred_element_type=f32)

    iota_v = jax.lax.broadcasted_iota(jnp.int32, (R, V), 1)
    oh_x = (idx[:, None] == iota_v).astype(bf16)
    x = jnp.dot(oh_x, emb_s[...], preferred_element_type=f32)

    iota_b = jax.lax.broadcasted_iota(jnp.int32, (R, B), 1)
    oh_c = (ctx[:, None] == iota_b).astype(bf16)
    tc = jnp.dot(oh_c, tv_s[...], preferred_element_type=f32)

    wz_b = bias_ref[0, :H]
    wr_b = bias_ref[1, :H]
    wh_b = bias_ref[2, :H]
    w_b = bias_ref[3, :H]
    ui_b = bias_ref[4, :H]
    u_b = bias_ref[5, :H]
    uo_row = bias_ref[6, :H]
    uo_b = bias_ref[7, 0]

    r1 = _dgt(x, wr_s[...]) + wr_b[None, :]
    r1_full = jnp.dot(segt_s[...], r1.astype(bf16),
                      preferred_element_type=f32)
    r2 = _dgt(x2b, ur_s[...])
    g = jax.nn.sigmoid(r1_full + r2)
    p = g.astype(bf16) * x2b
    sum_gated = jnp.dot(seg_s[...], p, preferred_element_type=f32)

    z = jax.nn.sigmoid(_dgt(x, wz_s[:, :H]) + _dgt(sum_h, wz_s[:, H:])
                       + wz_b[None, :])
    pre_h = jnp.tanh(_dgt(x, wh_s[:, :H]) + _dgt(sum_gated, wh_s[:, H:])
                     + wh_b[None, :])
    new_h = (1.0 - z) * sum_h + z * pre_h

    wh_act = jax.nn.relu(_dgt(new_h, w_s[:, :H]) + _dgt(tc, w_s[:, H:])
                         + w_b[None, :])
    word = _dgt(wh_act, wo_s[...])
    out_ref[:, :V] = word + bias_ref[8:8 + (V // H), :].reshape(1, V)

    sh = jax.nn.relu(_dgt(x, ui_s[:, :H]) + _dgt(sum_h, ui_s[:, H:])
                     + ui_b[None, :])
    sh2 = jax.nn.relu(_dgt(sh, u_s[:, :H]) + _dgt(tc, u_s[:, H:])
                      + u_b[None, :])
    stop = jnp.sum(sh2 * uo_row[None, :], axis=1, keepdims=True) + uo_b
    out_ref[:, V:] = stop


@jax.jit
def _run(cur_x_idx, contexts, cur_h_nei, tree_vecs, emb, Wz_w, Wz_b, Wr_w,
         Wr_b, Ur_w, Wh_w, Wh_b, W_w, W_b, Wo_w, Wo_b, Ui_w, Ui_b, U_w, U_b,
         Uo_w, Uo_b):
    idx2 = cur_x_idx.astype(jnp.int32).reshape(NB, 1, R)
    ctx2 = contexts.astype(jnp.int32).reshape(NB, 1, R)

    bias = jnp.stack([
        Wz_b, Wr_b, Wh_b, W_b, Ui_b, U_b, Uo_w[0, :],
        jnp.full((H,), Uo_b[0], f32),
    ], axis=0)
    bias = jnp.concatenate([bias, Wo_b.reshape(V // H, H)], axis=0)

    full = lambda shape: pl.BlockSpec(shape, lambda i: (0,) * len(shape))
    grid = (NB,)
    in_specs = [
        pl.BlockSpec((1, 1, R), lambda i: (i, 0, 0)),
        pl.BlockSpec((1, 1, R), lambda i: (i, 0, 0)),
        pl.BlockSpec((R, MAXN, H), lambda i: (i, 0, 0)),
        full((B, L)),
        full((V, H)),
        full((H, 2 * H)),
        full((H, H)),
        full((H, H)),
        full((H, 2 * H)),
        full((H, H + L)),
        full((V, H)),
        full((H, 2 * H)),
        full((H, H + L)),
        full((8 + V // H, H)),
    ]
    out_specs = pl.BlockSpec((R, V + 1), lambda i: (i, 0))
    out = pl.pallas_call(
        _body,
        grid=grid,
        in_specs=in_specs,
        out_specs=out_specs,
        out_shape=jax.ShapeDtypeStruct((T, V + 1), f32),
        scratch_shapes=[
            pltpu.VMEM((H, 2 * H), bf16),
            pltpu.VMEM((H, H), bf16),
            pltpu.VMEM((H, H), bf16),
            pltpu.VMEM((H, 2 * H), bf16),
            pltpu.VMEM((H, H + L), bf16),
            pltpu.VMEM((V, H), bf16),
            pltpu.VMEM((H, 2 * H), bf16),
            pltpu.VMEM((H, H + L), bf16),
            pltpu.VMEM((V, H), bf16),
            pltpu.VMEM((B, L), bf16),
            pltpu.VMEM((R, RN), bf16),
            pltpu.VMEM((RN, R), bf16),
        ],
    )(idx2, ctx2, cur_h_nei, tree_vecs, emb, Wz_w, Wr_w, Ur_w, Wh_w, W_w,
      Wo_w, Ui_w, U_w, bias)
    return out


def kernel(cur_x_idx, contexts, cur_h_nei, tree_vecs, emb, Wz_w, Wz_b, Wr_w,
           Wr_b, Ur_w, Wh_w, Wh_b, W_w, W_b, Wo_w, Wo_b, Ui_w, Ui_b, U_w,
           U_b, Uo_w, Uo_b):
    return _run(cur_x_idx, contexts, cur_h_nei, tree_vecs, emb, Wz_w, Wz_b,
                Wr_w, Wr_b, Ur_w, Wh_w, Wh_b, W_w, W_b, Wo_w, Wo_b, Ui_w,
                Ui_b, U_w, U_b, Uo_w, Uo_b)
